# baseline (speedup 1.0000x reference)
import jax
import jax.numpy as jnp
from jax.experimental import pallas as pl
from jax.experimental.pallas import tpu as pltpu

B, C, H, W = 32, 768, 24, 24
HW = H * W
BB = 8
SP = 2
V = 3


def _body(idx_ref, emb_ref, x_ref, o_ref):
    b0 = pl.program_id(0) * BB
    rows = emb_ref[...]
    viota = jax.lax.broadcasted_iota(jnp.int32, (V, 1), 0)
    for bb in range(BB):
        v = idx_ref[b0 + bb]
        bias = jnp.sum(rows * (viota == v).astype(jnp.float32), axis=0,
                       keepdims=True)
        o_ref[bb] = x_ref[bb] + bias


def kernel(x, anatomy_idx, emb_table):
    xt = jnp.transpose(x, (0, 2, 3, 1)).reshape(B, HW, C)
    out = pl.pallas_call(
        _body,
        grid=(B // BB, SP),
        in_specs=[
            pl.BlockSpec(memory_space=pltpu.SMEM),
            pl.BlockSpec(memory_space=pltpu.VMEM),
            pl.BlockSpec((BB, HW // SP, C), lambda b, j: (b, j, 0)),
        ],
        out_specs=pl.BlockSpec((BB, HW // SP, C), lambda b, j: (b, j, 0)),
        out_shape=jax.ShapeDtypeStruct((B, HW, C), jnp.float32),
    )(anatomy_idx.astype(jnp.int32), emb_table, xt)
    return jnp.transpose(out.reshape(B, H, W, C), (0, 3, 1, 2))


# speedup vs baseline: 1.0350x; 1.0350x over previous
import jax
import jax.numpy as jnp
from jax.experimental import pallas as pl
from jax.experimental.pallas import tpu as pltpu

B, C, H, W = 32, 768, 24, 24
HW = H * W
BB = 8
V = 3


def _body(idx_ref, emb_ref, x_ref, o_ref):
    b0 = pl.program_id(0) * BB
    rows = emb_ref[...]
    viota = jax.lax.broadcasted_iota(jnp.int32, (V, 1), 0)
    for bb in range(BB):
        v = idx_ref[b0 + bb]
        bias = jnp.sum(rows * (viota == v).astype(jnp.float32), axis=0,
                       keepdims=True)
        o_ref[bb] = x_ref[bb] + bias


def kernel(x, anatomy_idx, emb_table):
    xt = jnp.transpose(x, (0, 2, 3, 1)).reshape(B, HW, C)
    out = pl.pallas_call(
        _body,
        grid=(B // BB,),
        in_specs=[
            pl.BlockSpec(memory_space=pltpu.SMEM),
            pl.BlockSpec(memory_space=pltpu.VMEM),
            pl.BlockSpec((BB, HW, C), lambda b: (b, 0, 0)),
        ],
        out_specs=pl.BlockSpec((BB, HW, C), lambda b: (b, 0, 0)),
        out_shape=jax.ShapeDtypeStruct((B, HW, C), jnp.float32),
    )(anatomy_idx.astype(jnp.int32), emb_table, xt)
    return jnp.transpose(out.reshape(B, H, W, C), (0, 3, 1, 2))
